# Initial kernel scaffold; baseline (speedup 1.0000x reference)
import jax
import jax.numpy as jnp
from jax.experimental import pallas as pl


def kernel(x, edge_index, W1, b1, W2, b2, Wout, bout):
    raise NotImplementedError("write your pallas kernel here")


# speedup vs baseline: 9.5010x; 9.5010x over previous
import functools

import jax
import jax.numpy as jnp
from jax import lax
from jax.experimental import pallas as pl
from jax.experimental.pallas import tpu as pltpu
import jax.experimental.pallas.tpu_sc as plsc

NN = 50000
EE = 800000
NP = 50176
EP = 819200
NC, NS = 2, 16
CHUNK = 128
NCHUNKS = EP // CHUNK
CPW = NCHUNKS // (NC * NS)
CPS = NCHUNKS // NS
RPT = NP // NS
BLK = 1024

_MESH = plsc.VectorSubcoreMesh(core_axis_name="c", subcore_axis_name="s",
                               num_cores=NC, num_subcores=NS)
_PARAMS = pltpu.CompilerParams(use_tc_tiling_on_sc=False,
                               needs_layout_passes=False)


def _rsqrt_newton(d):
    i = plsc.bitcast(d, jnp.int32)
    g = plsc.bitcast(jnp.int32(0x5F3759DF) - (i >> 1), jnp.float32)
    for _ in range(3):
        g = g * (1.5 - 0.5 * d * g * g)
    return g


@functools.partial(
    pl.kernel,
    out_type=[
        jax.ShapeDtypeStruct((NC, NP, 4), jnp.float32),
        jax.ShapeDtypeStruct((NC, NP, 4), jnp.float32),
        jax.ShapeDtypeStruct((NC, NP, 4), jnp.float32),
    ],
    mesh=_MESH,
    compiler_params=_PARAMS,
    scratch_types=[
        pltpu.VMEM((CPS, CHUNK), jnp.int32),
        pltpu.VMEM((CHUNK, 4), jnp.float32),
        pltpu.VMEM((RPT, 4), jnp.float32),
        pltpu.VMEM((RPT, 4), jnp.float32),
        pltpu.VMEM((CHUNK, 4), jnp.float32),
        pltpu.VMEM_SHARED((NP, 4), jnp.float32),
        pltpu.SemaphoreType.DMA,
    ],
)
def _sc_mega(dst_c, src_c, x4, ones_n, ones_c, z4,
             deg_o, y1_o, s1_o,
             idx, vals, degb, xb, rows, acc, sem):
    c = lax.axis_index("c")
    s = lax.axis_index("s")
    r0 = s * RPT
    pltpu.sync_copy(ones_n.at[pl.ds(r0, RPT)], acc.at[pl.ds(r0, RPT)])
    pltpu.sync_copy(ones_c, vals)
    pltpu.sync_copy(dst_c.at[s], idx)
    pltpu.sync_copy(x4.at[pl.ds(r0, RPT)], xb)
    plsc.subcore_barrier()

    def hstep(j, carry):
        pltpu.sync_copy(vals, acc.at[idx.at[j]], add=True)
        return carry

    lax.fori_loop(0, CPS, hstep, 0)
    plsc.subcore_barrier()
    pltpu.sync_copy(src_c.at[s, pl.ds(c * CPW, CPW)], idx.at[pl.ds(0, CPW)])
    pltpu.sync_copy(dst_c.at[s, pl.ds(c * CPW, CPW)],
                    idx.at[pl.ds(CPW, CPW)])

    pltpu.sync_copy(acc.at[pl.ds(r0, RPT)], degb)
    pltpu.sync_copy(degb, deg_o.at[c, pl.ds(r0, RPT)])
    lanes = lax.iota(jnp.int32, 16)

    def ystep(i, carry):
        flat = i * 16 + lanes
        r = flat >> 2
        q = flat & 3
        d = plsc.load_gather(degb, [r, q])
        xv = plsc.load_gather(xb, [r, q])
        plsc.store_scatter(xb, [r, q], _rsqrt_newton(d) * xv)
        return carry

    lax.fori_loop(0, RPT * 4 // 16, ystep, 0)
    pltpu.sync_copy(xb, y1_o.at[c, pl.ds(r0, RPT)])
    pltpu.sync_copy(z4.at[pl.ds(r0, RPT)], acc.at[pl.ds(r0, RPT)])
    plsc.subcore_barrier()

    def astep(j, carry):
        pltpu.async_copy(y1_o.at[c].at[idx.at[j]], rows, sem).wait()
        pltpu.sync_copy(rows, acc.at[idx.at[CPW + j]], add=True)
        return carry

    lax.fori_loop(0, CPW, astep, 0)
    plsc.subcore_barrier()
    pltpu.sync_copy(acc.at[pl.ds(r0, RPT)], s1_o.at[c, pl.ds(r0, RPT)])


@functools.partial(
    pl.kernel,
    out_type=jax.ShapeDtypeStruct((8, NP, 8), jnp.float32),
    mesh=_MESH,
    compiler_params=_PARAMS,
    scratch_types=[
        pltpu.VMEM((CPS, CHUNK), jnp.int32),
        pltpu.VMEM((CPS, CHUNK), jnp.int32),
        pltpu.VMEM((CHUNK, 8), jnp.float32),
        pltpu.VMEM_SHARED((NP, 8), jnp.float32),
        pltpu.SemaphoreType.DMA,
    ],
)
def _sc_agg16(src_c, dst_c, tabs, z8, out, sidx, didx, rows, acc, sem):
    c = lax.axis_index("c")
    s = lax.axis_index("s")
    r0 = s * RPT
    pltpu.sync_copy(src_c.at[s], sidx)
    pltpu.sync_copy(dst_c.at[s], didx)
    for part in range(4):
        t = part * 2 + c
        pltpu.sync_copy(z8.at[pl.ds(r0, RPT)], acc.at[pl.ds(r0, RPT)])
        plsc.subcore_barrier()

        def astep(j, carry):
            pltpu.async_copy(tabs.at[t].at[sidx.at[j]], rows, sem).wait()
            pltpu.sync_copy(rows, acc.at[didx.at[j]], add=True)
            return carry

        lax.fori_loop(0, CPS, astep, 0)
        plsc.subcore_barrier()
        pltpu.sync_copy(acc.at[pl.ds(r0, RPT)], out.at[t, pl.ds(r0, RPT)])


def _tc_b_body(degp, s1p, y1p, W1, b1, tabs_o, dis_o):
    dis = lax.rsqrt(degp[0, :, 0:1])
    agg = dis * (s1p[0] + s1p[1] + y1p[0])
    h = jnp.maximum(jnp.dot(agg, W1[...], preferred_element_type=jnp.float32)
                    + b1[...], 0.0)
    y2 = dis * h
    dis_o[...] = dis
    for k in range(8):
        tabs_o[k] = y2[:, 8 * k:8 * (k + 1)]


_tc_b = pl.pallas_call(
    _tc_b_body,
    grid=(NP // BLK,),
    in_specs=[
        pl.BlockSpec((NC, BLK, 4), lambda i: (0, i, 0)),
        pl.BlockSpec((NC, BLK, 4), lambda i: (0, i, 0)),
        pl.BlockSpec((NC, BLK, 4), lambda i: (0, i, 0)),
        pl.BlockSpec((4, 64), lambda i: (0, 0)),
        pl.BlockSpec((1, 64), lambda i: (0, 0)),
    ],
    out_specs=[
        pl.BlockSpec((8, BLK, 8), lambda i: (0, i, 0)),
        pl.BlockSpec((BLK, 1), lambda i: (i, 0)),
    ],
    out_shape=[
        jax.ShapeDtypeStruct((8, NP, 8), jnp.float32),
        jax.ShapeDtypeStruct((NP, 1), jnp.float32),
    ],
)


def _tc_c_body(s2q, tabs, dis, W2, b2, Wop, bop, pred_o):
    dis_v = dis[...]
    agg = jnp.concatenate(
        [dis_v * (s2q[k] + tabs[k]) for k in range(8)], axis=1)
    h = jnp.maximum(jnp.dot(agg, W2[...], preferred_element_type=jnp.float32)
                    + b2[...], 0.0)
    pred_o[...] = (jnp.dot(h, Wop[...], preferred_element_type=jnp.float32)
                   + bop[...])


_tc_c = pl.pallas_call(
    _tc_c_body,
    grid=(NP // BLK,),
    in_specs=[
        pl.BlockSpec((8, BLK, 8), lambda i: (0, i, 0)),
        pl.BlockSpec((8, BLK, 8), lambda i: (0, i, 0)),
        pl.BlockSpec((BLK, 1), lambda i: (i, 0)),
        pl.BlockSpec((64, 64), lambda i: (0, 0)),
        pl.BlockSpec((1, 64), lambda i: (0, 0)),
        pl.BlockSpec((64, 8), lambda i: (0, 0)),
        pl.BlockSpec((1, 8), lambda i: (0, 0)),
    ],
    out_specs=pl.BlockSpec((BLK, 8), lambda i: (i, 0)),
    out_shape=jax.ShapeDtypeStruct((NP, 8), jnp.float32),
)


def kernel(x, edge_index, W1, b1, W2, b2, Wout, bout):
    src = edge_index[0]
    dst = edge_index[1]
    pad = jnp.full((EP - EE,), NN, jnp.int32)
    src_f = jnp.concatenate([src, pad])
    dst_f = jnp.concatenate([dst, pad])
    src_c = src_f.reshape(NS, CPS, CHUNK)
    dst_c = dst_f.reshape(NS, CPS, CHUNK)
    x4 = jnp.pad(x, ((0, NP - NN), (0, 0)))
    z4 = jnp.zeros((NP, 4), jnp.float32)
    z8 = jnp.zeros((NP, 8), jnp.float32)
    ones_n = jnp.ones((NP, 4), jnp.float32)
    ones_c = jnp.ones((CHUNK, 4), jnp.float32)
    Wop = jnp.pad(Wout, ((0, 0), (0, 5)))
    bop = jnp.pad(bout, (0, 5)).reshape(1, 8)

    degp, y1p, s1p = _sc_mega(dst_c, src_c, x4, ones_n, ones_c, z4)
    tabs, dis = _tc_b(degp, s1p, y1p, W1, b1.reshape(1, 64))
    s2q = _sc_agg16(src_c, dst_c, tabs, z8)
    pred = _tc_c(s2q, tabs, dis, W2, b2.reshape(1, 64), Wop, bop)
    return pred[:NN, :3]


# speedup vs baseline: 11.1212x; 1.1705x over previous
import functools

import jax
import jax.numpy as jnp
from jax import lax
from jax.experimental import pallas as pl
from jax.experimental.pallas import tpu as pltpu
import jax.experimental.pallas.tpu_sc as plsc

NN = 50000
EE = 800000
NP = 50176
EP = 819200
NC, NS = 2, 16
CHUNK = 128
NCHUNKS = EP // CHUNK
CPW = NCHUNKS // (NC * NS)
CPS = NCHUNKS // NS
NBUF = 8
NBUF_M = 4
RPT = NP // NS
BLK = 1024

_MESH = plsc.VectorSubcoreMesh(core_axis_name="c", subcore_axis_name="s",
                               num_cores=NC, num_subcores=NS)
_PARAMS = pltpu.CompilerParams(use_tc_tiling_on_sc=False,
                               needs_layout_passes=False)


def _rsqrt_newton(d):
    i = plsc.bitcast(d, jnp.int32)
    g = plsc.bitcast(jnp.int32(0x5F3759DF) - (i >> 1), jnp.float32)
    for _ in range(3):
        g = g * (1.5 - 0.5 * d * g * g)
    return g


@functools.partial(
    pl.kernel,
    out_type=[
        jax.ShapeDtypeStruct((NC, NP, 4), jnp.float32),
        jax.ShapeDtypeStruct((NC, NP, 4), jnp.float32),
        jax.ShapeDtypeStruct((NC, NP, 4), jnp.float32),
    ],
    mesh=_MESH,
    compiler_params=_PARAMS,
    scratch_types=[
        pltpu.VMEM((CPS, CHUNK), jnp.int32),
        pltpu.VMEM((CHUNK, 4), jnp.float32),
        pltpu.VMEM((RPT, 4), jnp.float32),
        pltpu.VMEM((RPT, 4), jnp.float32),
        pltpu.VMEM((CHUNK, 4), jnp.float32),
        pltpu.VMEM_SHARED((NP, 4), jnp.float32),
        pltpu.SemaphoreType.DMA,
    ],
)
def _sc_mega(dst_c, src_c, x4, ones_n, ones_c, z4,
             deg_o, y1_o, s1_o,
             idx, vals, degb, xb, rows, acc, sem):
    c = lax.axis_index("c")
    s = lax.axis_index("s")
    r0 = s * RPT
    pltpu.sync_copy(ones_n.at[pl.ds(r0, RPT)], acc.at[pl.ds(r0, RPT)])
    pltpu.sync_copy(ones_c, vals)
    pltpu.sync_copy(dst_c.at[s], idx)
    pltpu.sync_copy(x4.at[pl.ds(r0, RPT)], xb)
    plsc.subcore_barrier()

    def hstep(j, carry):
        pltpu.sync_copy(vals, acc.at[idx.at[j]], add=True)
        return carry

    lax.fori_loop(0, CPS, hstep, 0)
    plsc.subcore_barrier()
    pltpu.sync_copy(src_c.at[s, pl.ds(c * CPW, CPW)], idx.at[pl.ds(0, CPW)])
    pltpu.sync_copy(dst_c.at[s, pl.ds(c * CPW, CPW)],
                    idx.at[pl.ds(CPW, CPW)])

    pltpu.sync_copy(acc.at[pl.ds(r0, RPT)], degb)
    pltpu.sync_copy(degb, deg_o.at[c, pl.ds(r0, RPT)])
    lanes = lax.iota(jnp.int32, 16)

    def ystep(i, carry):
        flat = i * 16 + lanes
        r = flat >> 2
        q = flat & 3
        d = plsc.load_gather(degb, [r, q])
        xv = plsc.load_gather(xb, [r, q])
        plsc.store_scatter(xb, [r, q], _rsqrt_newton(d) * xv)
        return carry

    lax.fori_loop(0, RPT * 4 // 16, ystep, 0)
    pltpu.sync_copy(xb, y1_o.at[c, pl.ds(r0, RPT)])
    pltpu.sync_copy(z4.at[pl.ds(r0, RPT)], acc.at[pl.ds(r0, RPT)])
    plsc.subcore_barrier()

    def astep(j, carry):
        pltpu.async_copy(y1_o.at[c].at[idx.at[j]], rows, sem).wait()
        pltpu.sync_copy(rows, acc.at[idx.at[CPW + j]], add=True)
        return carry

    lax.fori_loop(0, CPW, astep, 0)
    plsc.subcore_barrier()
    pltpu.sync_copy(acc.at[pl.ds(r0, RPT)], s1_o.at[c, pl.ds(r0, RPT)])


@functools.partial(
    pl.kernel,
    out_type=jax.ShapeDtypeStruct((8, NP, 8), jnp.float32),
    mesh=_MESH,
    compiler_params=_PARAMS,
    scratch_types=[
        pltpu.VMEM((CPS, CHUNK), jnp.int32),
        pltpu.VMEM((CPS, CHUNK), jnp.int32),
        pltpu.VMEM((2, CHUNK, 8), jnp.float32),
        pltpu.VMEM_SHARED((NP, 8), jnp.float32),
        pltpu.SemaphoreType.DMA,
    ],
)
def _sc_agg16(src_c, dst_c, tabs, z8, out, sidx, didx, rows, acc, gsem):
    c = lax.axis_index("c")
    s = lax.axis_index("s")
    r0 = s * RPT
    pltpu.sync_copy(src_c.at[s], sidx)
    pltpu.sync_copy(dst_c.at[s], didx)
    for part in range(4):
        t = part * 2 + c
        pltpu.sync_copy(z8.at[pl.ds(r0, RPT)], acc.at[pl.ds(r0, RPT)])
        plsc.subcore_barrier()

        def astep(jo, carry):
            base = jo * 2
            gd = [pltpu.async_copy(tabs.at[t].at[sidx.at[base + b]],
                                   rows.at[b], gsem) for b in range(2)]
            for d in gd:
                d.wait()
            for b in range(2):
                pltpu.sync_copy(rows.at[b], acc.at[didx.at[base + b]],
                                add=True)
            return carry

        lax.fori_loop(0, CPS // 2, astep, 0)
        plsc.subcore_barrier()
        pltpu.sync_copy(acc.at[pl.ds(r0, RPT)], out.at[t, pl.ds(r0, RPT)])


def _tc_b_body(degp, s1p, y1p, W1, b1, tabs_o, dis_o):
    dis = lax.rsqrt(degp[0, :, 0:1])
    agg = dis * (s1p[0] + s1p[1] + y1p[0])
    h = jnp.maximum(jnp.dot(agg, W1[...], preferred_element_type=jnp.float32)
                    + b1[...], 0.0)
    y2 = dis * h
    dis_o[...] = dis
    for k in range(8):
        tabs_o[k] = y2[:, 8 * k:8 * (k + 1)]


_tc_b = pl.pallas_call(
    _tc_b_body,
    grid=(NP // BLK,),
    in_specs=[
        pl.BlockSpec((NC, BLK, 4), lambda i: (0, i, 0)),
        pl.BlockSpec((NC, BLK, 4), lambda i: (0, i, 0)),
        pl.BlockSpec((NC, BLK, 4), lambda i: (0, i, 0)),
        pl.BlockSpec((4, 64), lambda i: (0, 0)),
        pl.BlockSpec((1, 64), lambda i: (0, 0)),
    ],
    out_specs=[
        pl.BlockSpec((8, BLK, 8), lambda i: (0, i, 0)),
        pl.BlockSpec((BLK, 1), lambda i: (i, 0)),
    ],
    out_shape=[
        jax.ShapeDtypeStruct((8, NP, 8), jnp.float32),
        jax.ShapeDtypeStruct((NP, 1), jnp.float32),
    ],
)


def _tc_c_body(s2q, tabs, dis, W2, b2, Wop, bop, pred_o):
    dis_v = dis[...]
    agg = jnp.concatenate(
        [dis_v * (s2q[k] + tabs[k]) for k in range(8)], axis=1)
    h = jnp.maximum(jnp.dot(agg, W2[...], preferred_element_type=jnp.float32)
                    + b2[...], 0.0)
    pred_o[...] = (jnp.dot(h, Wop[...], preferred_element_type=jnp.float32)
                   + bop[...])


_tc_c = pl.pallas_call(
    _tc_c_body,
    grid=(NP // BLK,),
    in_specs=[
        pl.BlockSpec((8, BLK, 8), lambda i: (0, i, 0)),
        pl.BlockSpec((8, BLK, 8), lambda i: (0, i, 0)),
        pl.BlockSpec((BLK, 1), lambda i: (i, 0)),
        pl.BlockSpec((64, 64), lambda i: (0, 0)),
        pl.BlockSpec((1, 64), lambda i: (0, 0)),
        pl.BlockSpec((64, 8), lambda i: (0, 0)),
        pl.BlockSpec((1, 8), lambda i: (0, 0)),
    ],
    out_specs=pl.BlockSpec((BLK, 8), lambda i: (i, 0)),
    out_shape=jax.ShapeDtypeStruct((NP, 8), jnp.float32),
)


def kernel(x, edge_index, W1, b1, W2, b2, Wout, bout):
    src = edge_index[0]
    dst = edge_index[1]
    pad = jnp.full((EP - EE,), NN, jnp.int32)
    src_f = jnp.concatenate([src, pad])
    dst_f = jnp.concatenate([dst, pad])
    src_c = src_f.reshape(NS, CPS, CHUNK)
    dst_c = dst_f.reshape(NS, CPS, CHUNK)
    x4 = jnp.pad(x, ((0, NP - NN), (0, 0)))
    z4 = jnp.zeros((NP, 4), jnp.float32)
    z8 = jnp.zeros((NP, 8), jnp.float32)
    ones_n = jnp.ones((NP, 4), jnp.float32)
    ones_c = jnp.ones((CHUNK, 4), jnp.float32)
    Wop = jnp.pad(Wout, ((0, 0), (0, 5)))
    bop = jnp.pad(bout, (0, 5)).reshape(1, 8)

    degp, y1p, s1p = _sc_mega(dst_c, src_c, x4, ones_n, ones_c, z4)
    tabs, dis = _tc_b(degp, s1p, y1p, W1, b1.reshape(1, 64))
    s2q = _sc_agg16(src_c, dst_c, tabs, z8)
    pred = _tc_c(s2q, tabs, dis, W2, b2.reshape(1, 64), Wop, bop)
    return pred[:NN, :3]


# speedup vs baseline: 13.1232x; 1.1800x over previous
import functools

import jax
import jax.numpy as jnp
from jax import lax
from jax.experimental import pallas as pl
from jax.experimental.pallas import tpu as pltpu
import jax.experimental.pallas.tpu_sc as plsc

NN = 50000
EE = 800000
NP = 50176
EP = 819200
NC, NS = 2, 16
CHUNK = 128
NCHUNKS = EP // CHUNK
CPW = NCHUNKS // (NC * NS)
CPS = NCHUNKS // NS
RING = 8
NBA = RING // 2
HCPS = CPS // 2
RPT = NP // NS
BLK = 1024

_MESH = plsc.VectorSubcoreMesh(core_axis_name="c", subcore_axis_name="s",
                               num_cores=NC, num_subcores=NS)
_PARAMS = pltpu.CompilerParams(use_tc_tiling_on_sc=False,
                               needs_layout_passes=False)


def _rsqrt_newton(d):
    i = plsc.bitcast(d, jnp.int32)
    g = plsc.bitcast(jnp.int32(0x5F3759DF) - (i >> 1), jnp.float32)
    for _ in range(3):
        g = g * (1.5 - 0.5 * d * g * g)
    return g


@functools.partial(
    pl.kernel,
    out_type=[
        jax.ShapeDtypeStruct((NC, NP, 4), jnp.float32),
        jax.ShapeDtypeStruct((NC, NP, 4), jnp.float32),
        jax.ShapeDtypeStruct((NC, NP, 4), jnp.float32),
    ],
    mesh=_MESH,
    compiler_params=_PARAMS,
    scratch_types=[
        pltpu.VMEM((CPS, CHUNK), jnp.int32),
        pltpu.VMEM((RPT, 4), jnp.float32),
        pltpu.VMEM((RPT, 4), jnp.float32),
        pltpu.VMEM((CHUNK, 4), jnp.float32),
        pltpu.VMEM_SHARED((NP, 4), jnp.float32),
        pltpu.SemaphoreType.DMA,
    ],
)
def _sc_mega(dst_c, src_c, x4, ones_n, ones_c, z4,
             deg_o, y1_o, s1_o,
             idx, degb, xb, rows, acc, sem):
    c = lax.axis_index("c")
    s = lax.axis_index("s")
    r0 = s * RPT
    pltpu.sync_copy(ones_n.at[pl.ds(r0, RPT)], acc.at[pl.ds(r0, RPT)])
    pltpu.sync_copy(ones_c, rows)
    pltpu.sync_copy(dst_c.at[s], idx)
    pltpu.sync_copy(x4.at[pl.ds(r0, RPT)], xb)
    plsc.subcore_barrier()

    def hstep(j, carry):
        pltpu.sync_copy(rows, acc.at[idx.at[j]], add=True)
        return carry

    lax.fori_loop(0, CPS, hstep, 0)
    plsc.subcore_barrier()
    pltpu.sync_copy(src_c.at[s, pl.ds(c * CPW, CPW)], idx.at[pl.ds(0, CPW)])
    pltpu.sync_copy(dst_c.at[s, pl.ds(c * CPW, CPW)],
                    idx.at[pl.ds(CPW, CPW)])

    pltpu.sync_copy(acc.at[pl.ds(r0, RPT)], degb)
    pltpu.sync_copy(degb, deg_o.at[c, pl.ds(r0, RPT)])
    lanes = lax.iota(jnp.int32, 16)

    def ystep(i, carry):
        flat = i * 16 + lanes
        r = flat >> 2
        q = flat & 3
        d = plsc.load_gather(degb, [r, q])
        xv = plsc.load_gather(xb, [r, q])
        plsc.store_scatter(xb, [r, q], _rsqrt_newton(d) * xv)
        return carry

    lax.fori_loop(0, RPT * 4 // 16, ystep, 0)
    pltpu.sync_copy(xb, y1_o.at[c, pl.ds(r0, RPT)])
    pltpu.sync_copy(z4.at[pl.ds(r0, RPT)], acc.at[pl.ds(r0, RPT)])
    plsc.subcore_barrier()

    def astep(j, carry):
        pltpu.async_copy(y1_o.at[c].at[idx.at[j]], rows, sem).wait()
        pltpu.sync_copy(rows, acc.at[idx.at[CPW + j]], add=True)
        return carry

    lax.fori_loop(0, CPW, astep, 0)
    plsc.subcore_barrier()
    pltpu.sync_copy(acc.at[pl.ds(r0, RPT)], s1_o.at[c, pl.ds(r0, RPT)])


@functools.partial(
    pl.kernel,
    out_type=jax.ShapeDtypeStruct((8, NP, 8), jnp.float32),
    mesh=_MESH,
    compiler_params=_PARAMS,
    scratch_types=[
        pltpu.VMEM((CPS, CHUNK), jnp.int32),
        pltpu.VMEM((HCPS, CHUNK), jnp.int32),
        pltpu.VMEM((RING, CHUNK, 8), jnp.float32),
        pltpu.VMEM_SHARED((NP, 8), jnp.float32),
        pltpu.SemaphoreType.DMA,
    ],
)
def _sc_agg16(src_c, dst_c, tabs, z8, out, sidx, didx, rows, acc, gsem):
    c = lax.axis_index("c")
    s = lax.axis_index("s")
    r0 = s * RPT
    pltpu.sync_copy(src_c.at[s], sidx)
    for part in range(4):
        t = part * 2 + c
        tab = tabs.at[t]
        pltpu.sync_copy(z8.at[pl.ds(r0, RPT)], acc.at[pl.ds(r0, RPT)])
        plsc.subcore_barrier()

        for b in range(NBA):
            pltpu.async_copy(tab.at[sidx.at[b]], rows.at[b], gsem)

        for h in range(2):
            pltpu.sync_copy(dst_c.at[s, pl.ds(h * HCPS, HCPS)], didx)

            def astep(g, carry):
                for ph in range(2):
                    loc = (2 * g + ph) * NBA
                    base = h * HCPS + loc
                    for b in range(NBA):
                        pltpu.make_async_copy(
                            tab.at[pl.ds(0, CHUNK)],
                            rows.at[ph * NBA + b], gsem).wait()
                    for b in range(NBA):
                        nj = lax.rem(base + NBA + b, CPS)
                        pltpu.async_copy(tab.at[sidx.at[nj]],
                                         rows.at[(1 - ph) * NBA + b], gsem)
                    for b in range(NBA):
                        pltpu.sync_copy(rows.at[ph * NBA + b],
                                        acc.at[didx.at[loc + b]], add=True)
                return carry

            lax.fori_loop(0, HCPS // (2 * NBA), astep, 0)
        for b in range(NBA):
            pltpu.make_async_copy(tab.at[pl.ds(0, CHUNK)],
                                  rows.at[b], gsem).wait()
        plsc.subcore_barrier()
        pltpu.sync_copy(acc.at[pl.ds(r0, RPT)], out.at[t, pl.ds(r0, RPT)])


def _tc_b_body(degp, s1p, y1p, W1, b1, tabs_o, dis_o):
    dis = lax.rsqrt(degp[0, :, 0:1])
    agg = dis * (s1p[0] + s1p[1] + y1p[0])
    h = jnp.maximum(jnp.dot(agg, W1[...], preferred_element_type=jnp.float32)
                    + b1[...], 0.0)
    y2 = dis * h
    dis_o[...] = dis
    for k in range(8):
        tabs_o[k] = y2[:, 8 * k:8 * (k + 1)]


_tc_b = pl.pallas_call(
    _tc_b_body,
    grid=(NP // BLK,),
    in_specs=[
        pl.BlockSpec((NC, BLK, 4), lambda i: (0, i, 0)),
        pl.BlockSpec((NC, BLK, 4), lambda i: (0, i, 0)),
        pl.BlockSpec((NC, BLK, 4), lambda i: (0, i, 0)),
        pl.BlockSpec((4, 64), lambda i: (0, 0)),
        pl.BlockSpec((1, 64), lambda i: (0, 0)),
    ],
    out_specs=[
        pl.BlockSpec((8, BLK, 8), lambda i: (0, i, 0)),
        pl.BlockSpec((BLK, 1), lambda i: (i, 0)),
    ],
    out_shape=[
        jax.ShapeDtypeStruct((8, NP, 8), jnp.float32),
        jax.ShapeDtypeStruct((NP, 1), jnp.float32),
    ],
)


def _tc_c_body(s2q, tabs, dis, W2, b2, Wop, bop, pred_o):
    dis_v = dis[...]
    agg = jnp.concatenate(
        [dis_v * (s2q[k] + tabs[k]) for k in range(8)], axis=1)
    h = jnp.maximum(jnp.dot(agg, W2[...], preferred_element_type=jnp.float32)
                    + b2[...], 0.0)
    pred_o[...] = (jnp.dot(h, Wop[...], preferred_element_type=jnp.float32)
                   + bop[...])


_tc_c = pl.pallas_call(
    _tc_c_body,
    grid=(NP // BLK,),
    in_specs=[
        pl.BlockSpec((8, BLK, 8), lambda i: (0, i, 0)),
        pl.BlockSpec((8, BLK, 8), lambda i: (0, i, 0)),
        pl.BlockSpec((BLK, 1), lambda i: (i, 0)),
        pl.BlockSpec((64, 64), lambda i: (0, 0)),
        pl.BlockSpec((1, 64), lambda i: (0, 0)),
        pl.BlockSpec((64, 8), lambda i: (0, 0)),
        pl.BlockSpec((1, 8), lambda i: (0, 0)),
    ],
    out_specs=pl.BlockSpec((BLK, 8), lambda i: (i, 0)),
    out_shape=jax.ShapeDtypeStruct((NP, 8), jnp.float32),
)


def kernel(x, edge_index, W1, b1, W2, b2, Wout, bout):
    src = edge_index[0]
    dst = edge_index[1]
    pad = jnp.full((EP - EE,), NN, jnp.int32)
    src_f = jnp.concatenate([src, pad])
    dst_f = jnp.concatenate([dst, pad])
    src_c = src_f.reshape(NS, CPS, CHUNK)
    dst_c = dst_f.reshape(NS, CPS, CHUNK)
    x4 = jnp.pad(x, ((0, NP - NN), (0, 0)))
    z4 = jnp.zeros((NP, 4), jnp.float32)
    z8 = jnp.zeros((NP, 8), jnp.float32)
    ones_n = jnp.ones((NP, 4), jnp.float32)
    ones_c = jnp.ones((CHUNK, 4), jnp.float32)
    Wop = jnp.pad(Wout, ((0, 0), (0, 5)))
    bop = jnp.pad(bout, (0, 5)).reshape(1, 8)

    degp, y1p, s1p = _sc_mega(dst_c, src_c, x4, ones_n, ones_c, z4)
    tabs, dis = _tc_b(degp, s1p, y1p, W1, b1.reshape(1, 64))
    s2q = _sc_agg16(src_c, dst_c, tabs, z8)
    pred = _tc_c(s2q, tabs, dis, W2, b2.reshape(1, 64), Wop, bop)
    return pred[:NN, :3]


# speedup vs baseline: 13.3827x; 1.0198x over previous
import functools

import jax
import jax.numpy as jnp
from jax import lax
from jax.experimental import pallas as pl
from jax.experimental.pallas import tpu as pltpu
import jax.experimental.pallas.tpu_sc as plsc

NN = 50000
EE = 800000
NP = 50176
EP = 819200
NC, NS = 2, 16
CHUNK = 128
NCHUNKS = EP // CHUNK
CPW = NCHUNKS // (NC * NS)
CPS = NCHUNKS // NS
RING = 8
NBA = RING // 2
HCPS = CPS // 2
RPT = NP // NS
BLK = 1024

_MESH = plsc.VectorSubcoreMesh(core_axis_name="c", subcore_axis_name="s",
                               num_cores=NC, num_subcores=NS)
_PARAMS = pltpu.CompilerParams(use_tc_tiling_on_sc=False,
                               needs_layout_passes=False)


def _rsqrt_newton(d):
    i = plsc.bitcast(d, jnp.int32)
    g = plsc.bitcast(jnp.int32(0x5F3759DF) - (i >> 1), jnp.float32)
    for _ in range(3):
        g = g * (1.5 - 0.5 * d * g * g)
    return g


@functools.partial(
    pl.kernel,
    out_type=[
        jax.ShapeDtypeStruct((NC, NP, 4), jnp.float32),
        jax.ShapeDtypeStruct((NC, NP, 4), jnp.float32),
        jax.ShapeDtypeStruct((NC, NP, 4), jnp.float32),
    ],
    mesh=_MESH,
    compiler_params=_PARAMS,
    scratch_types=[
        pltpu.VMEM((CPS, CHUNK), jnp.int32),
        pltpu.VMEM((RPT, 4), jnp.float32),
        pltpu.VMEM((RPT, 4), jnp.float32),
        pltpu.VMEM((4, CHUNK, 4), jnp.float32),
        pltpu.VMEM_SHARED((NP, 4), jnp.float32),
        pltpu.SemaphoreType.DMA,
    ],
)
def _sc_mega(dst_c, src_c, x4, ones_n, ones_c, z4,
             deg_o, y1_o, s1_o,
             idx, degb, xb, rows, acc, sem):
    c = lax.axis_index("c")
    s = lax.axis_index("s")
    r0 = s * RPT
    pltpu.sync_copy(ones_n.at[pl.ds(r0, RPT)], acc.at[pl.ds(r0, RPT)])
    pltpu.sync_copy(ones_c, rows.at[0])
    pltpu.sync_copy(dst_c.at[s], idx)
    pltpu.sync_copy(x4.at[pl.ds(r0, RPT)], xb)
    plsc.subcore_barrier()

    def hstep(j, carry):
        pltpu.sync_copy(rows.at[0], acc.at[idx.at[j]], add=True)
        return carry

    lax.fori_loop(0, CPS, hstep, 0)
    plsc.subcore_barrier()
    pltpu.sync_copy(src_c.at[s, pl.ds(c * CPW, CPW)], idx.at[pl.ds(0, CPW)])
    pltpu.sync_copy(dst_c.at[s, pl.ds(c * CPW, CPW)],
                    idx.at[pl.ds(CPW, CPW)])

    pltpu.sync_copy(acc.at[pl.ds(r0, RPT)], degb)
    pltpu.sync_copy(degb, deg_o.at[c, pl.ds(r0, RPT)])
    lanes = lax.iota(jnp.int32, 16)

    def ystep(i, carry):
        flat = i * 16 + lanes
        r = flat >> 2
        q = flat & 3
        d = plsc.load_gather(degb, [r, q])
        xv = plsc.load_gather(xb, [r, q])
        plsc.store_scatter(xb, [r, q], _rsqrt_newton(d) * xv)
        return carry

    lax.fori_loop(0, RPT * 4 // 16, ystep, 0)
    pltpu.sync_copy(xb, y1_o.at[c, pl.ds(r0, RPT)])
    pltpu.sync_copy(z4.at[pl.ds(r0, RPT)], acc.at[pl.ds(r0, RPT)])
    plsc.subcore_barrier()

    tab = y1_o.at[c]

    def astep(g, carry):
        base = g * 2
        gd = [pltpu.async_copy(tab.at[idx.at[base + b]], rows.at[b], sem)
              for b in range(2)]
        for d in gd:
            d.wait()
        for b in range(2):
            pltpu.sync_copy(rows.at[b], acc.at[idx.at[CPW + base + b]],
                            add=True)
        return carry

    lax.fori_loop(0, CPW // 2, astep, 0)
    plsc.subcore_barrier()
    pltpu.sync_copy(acc.at[pl.ds(r0, RPT)], s1_o.at[c, pl.ds(r0, RPT)])


@functools.partial(
    pl.kernel,
    out_type=jax.ShapeDtypeStruct((8, NP, 8), jnp.float32),
    mesh=_MESH,
    compiler_params=_PARAMS,
    scratch_types=[
        pltpu.VMEM((CPS, CHUNK), jnp.int32),
        pltpu.VMEM((HCPS, CHUNK), jnp.int32),
        pltpu.VMEM((RING, CHUNK, 8), jnp.float32),
        pltpu.VMEM_SHARED((NP, 8), jnp.float32),
        pltpu.SemaphoreType.DMA,
    ],
)
def _sc_agg16(src_c, dst_c, tabs, z8, out, sidx, didx, rows, acc, gsem):
    c = lax.axis_index("c")
    s = lax.axis_index("s")
    r0 = s * RPT
    pltpu.sync_copy(src_c.at[s], sidx)
    for part in range(4):
        t = part * 2 + c
        tab = tabs.at[t]
        pltpu.sync_copy(z8.at[pl.ds(r0, RPT)], acc.at[pl.ds(r0, RPT)])
        plsc.subcore_barrier()

        for b in range(NBA):
            pltpu.async_copy(tab.at[sidx.at[b]], rows.at[b], gsem)

        for h in range(2):
            pltpu.sync_copy(dst_c.at[s, pl.ds(h * HCPS, HCPS)], didx)

            def astep(g, carry):
                for ph in range(2):
                    loc = (2 * g + ph) * NBA
                    base = h * HCPS + loc
                    for b in range(NBA):
                        pltpu.make_async_copy(
                            tab.at[pl.ds(0, CHUNK)],
                            rows.at[ph * NBA + b], gsem).wait()
                    for b in range(NBA):
                        nj = lax.rem(base + NBA + b, CPS)
                        pltpu.async_copy(tab.at[sidx.at[nj]],
                                         rows.at[(1 - ph) * NBA + b], gsem)
                    for b in range(NBA):
                        pltpu.sync_copy(rows.at[ph * NBA + b],
                                        acc.at[didx.at[loc + b]], add=True)
                return carry

            lax.fori_loop(0, HCPS // (2 * NBA), astep, 0)
        for b in range(NBA):
            pltpu.make_async_copy(tab.at[pl.ds(0, CHUNK)],
                                  rows.at[b], gsem).wait()
        plsc.subcore_barrier()
        pltpu.sync_copy(acc.at[pl.ds(r0, RPT)], out.at[t, pl.ds(r0, RPT)])


def _tc_b_body(degp, s1p, y1p, W1, b1, tabs_o, dis_o):
    dis = lax.rsqrt(degp[0, :, 0:1])
    agg = dis * (s1p[0] + s1p[1] + y1p[0])
    h = jnp.maximum(jnp.dot(agg, W1[...], preferred_element_type=jnp.float32)
                    + b1[...], 0.0)
    y2 = dis * h
    dis_o[...] = dis
    for k in range(8):
        tabs_o[k] = y2[:, 8 * k:8 * (k + 1)]


_tc_b = pl.pallas_call(
    _tc_b_body,
    grid=(NP // BLK,),
    in_specs=[
        pl.BlockSpec((NC, BLK, 4), lambda i: (0, i, 0)),
        pl.BlockSpec((NC, BLK, 4), lambda i: (0, i, 0)),
        pl.BlockSpec((NC, BLK, 4), lambda i: (0, i, 0)),
        pl.BlockSpec((4, 64), lambda i: (0, 0)),
        pl.BlockSpec((1, 64), lambda i: (0, 0)),
    ],
    out_specs=[
        pl.BlockSpec((8, BLK, 8), lambda i: (0, i, 0)),
        pl.BlockSpec((BLK, 1), lambda i: (i, 0)),
    ],
    out_shape=[
        jax.ShapeDtypeStruct((8, NP, 8), jnp.float32),
        jax.ShapeDtypeStruct((NP, 1), jnp.float32),
    ],
)


def _tc_c_body(s2q, tabs, dis, W2, b2, Wop, bop, pred_o):
    dis_v = dis[...]
    agg = jnp.concatenate(
        [dis_v * (s2q[k] + tabs[k]) for k in range(8)], axis=1)
    h = jnp.maximum(jnp.dot(agg, W2[...], preferred_element_type=jnp.float32)
                    + b2[...], 0.0)
    pred_o[...] = (jnp.dot(h, Wop[...], preferred_element_type=jnp.float32)
                   + bop[...])


_tc_c = pl.pallas_call(
    _tc_c_body,
    grid=(NP // BLK,),
    in_specs=[
        pl.BlockSpec((8, BLK, 8), lambda i: (0, i, 0)),
        pl.BlockSpec((8, BLK, 8), lambda i: (0, i, 0)),
        pl.BlockSpec((BLK, 1), lambda i: (i, 0)),
        pl.BlockSpec((64, 64), lambda i: (0, 0)),
        pl.BlockSpec((1, 64), lambda i: (0, 0)),
        pl.BlockSpec((64, 8), lambda i: (0, 0)),
        pl.BlockSpec((1, 8), lambda i: (0, 0)),
    ],
    out_specs=pl.BlockSpec((BLK, 8), lambda i: (i, 0)),
    out_shape=jax.ShapeDtypeStruct((NP, 8), jnp.float32),
)


def kernel(x, edge_index, W1, b1, W2, b2, Wout, bout):
    src = edge_index[0]
    dst = edge_index[1]
    pad = jnp.full((EP - EE,), NN, jnp.int32)
    src_f = jnp.concatenate([src, pad])
    dst_f = jnp.concatenate([dst, pad])
    src_c = src_f.reshape(NS, CPS, CHUNK)
    dst_c = dst_f.reshape(NS, CPS, CHUNK)
    x4 = jnp.pad(x, ((0, NP - NN), (0, 0)))
    z4 = jnp.zeros((NP, 4), jnp.float32)
    z8 = jnp.zeros((NP, 8), jnp.float32)
    ones_n = jnp.ones((NP, 4), jnp.float32)
    ones_c = jnp.ones((CHUNK, 4), jnp.float32)
    Wop = jnp.pad(Wout, ((0, 0), (0, 5)))
    bop = jnp.pad(bout, (0, 5)).reshape(1, 8)

    degp, y1p, s1p = _sc_mega(dst_c, src_c, x4, ones_n, ones_c, z4)
    tabs, dis = _tc_b(degp, s1p, y1p, W1, b1.reshape(1, 64))
    s2q = _sc_agg16(src_c, dst_c, tabs, z8)
    pred = _tc_c(s2q, tabs, dis, W2, b2.reshape(1, 64), Wop, bop)
    return pred[:NN, :3]
